# baseline (speedup 1.0000x reference)
import functools

import jax
import jax.numpy as jnp
from jax import lax
from jax.experimental import pallas as pl
from jax.experimental.pallas import tpu as pltpu
from jax.experimental.pallas import tpu_sc as plsc

_B = 16384
_F = 13
_D = 16
_NW = 32
_N = _B * _F
_RPW = _N // _NW
_ECH = 416
_NECH = _RPW // _ECH
_LCH = 832
_NLCH = _RPW // _LCH

_B_BLK = 512
_H = 208


def _sc_gather_body(pku, pks, ilu, ils, tabu, tabs, linu, lins,
                    gu, gs, lu, ls,
                    idx_v, rows_v, lidx_v, lval_v, sem):
    wid = lax.axis_index("s") * 2 + lax.axis_index("c")
    for c in range(_NECH):
        base = wid * _RPW + c * _ECH
        pltpu.sync_copy(pku.at[pl.ds(base, _ECH)], idx_v)
        pltpu.async_copy(tabu.at[idx_v], rows_v, sem).wait()
        pltpu.sync_copy(rows_v, gu.at[pl.ds(base, _ECH)])
        pltpu.sync_copy(pks.at[pl.ds(base, _ECH)], idx_v)
        pltpu.async_copy(tabs.at[idx_v], rows_v, sem).wait()
        pltpu.sync_copy(rows_v, gs.at[pl.ds(base, _ECH)])
    for c in range(_NLCH):
        base = wid * _RPW + c * _LCH
        pltpu.sync_copy(ilu.at[pl.ds(base, _LCH)], lidx_v)
        pltpu.async_copy(linu.at[lidx_v], lval_v, sem).wait()
        pltpu.sync_copy(lval_v, lu.at[pl.ds(base, _LCH)])
        pltpu.sync_copy(ils.at[pl.ds(base, _LCH)], lidx_v)
        pltpu.async_copy(lins.at[lidx_v], lval_v, sem).wait()
        pltpu.sync_copy(lval_v, ls.at[pl.ds(base, _LCH)])


def _sc_gather(pku, pks, ilu, ils, tabu, tabs, linu, lins):
    mesh = plsc.VectorSubcoreMesh(core_axis_name="c", subcore_axis_name="s")
    call = functools.partial(
        pl.kernel,
        mesh=mesh,
        out_type=[
            jax.ShapeDtypeStruct((_N, 128), jnp.float32),
            jax.ShapeDtypeStruct((_N, 128), jnp.float32),
            jax.ShapeDtypeStruct((_N,), jnp.float32),
            jax.ShapeDtypeStruct((_N,), jnp.float32),
        ],
        scratch_types=[
            pltpu.VMEM((_ECH,), jnp.int32),
            pltpu.VMEM((_ECH, 128), jnp.float32),
            pltpu.VMEM((_LCH,), jnp.int32),
            pltpu.VMEM((_LCH,), jnp.float32),
            pltpu.SemaphoreType.DMA,
        ],
    )(_sc_gather_body)
    return call(pku, pks, ilu, ils, tabu, tabs, linu, lins)


def _leaky(x):
    return jnp.where(x >= 0, x, 0.01 * x)


def _tc_body(gu, gs, remu, rems, ulg, slg, ux, sx,
             R, W0a, W0c, Wud, Wsd, uw, sw, uw2s, sw2s, udlw, sdlw,
             b0, W1, b1, W2, b2, W3, b3, W4, b4p, out):
    dot = lambda a, b: lax.dot_general(
        a, b, (((1,), (0,)), ((), ())), preferred_element_type=jnp.float32)
    ux_ = ux[...]
    sx_ = sx[...]
    R_ = R[...]
    lane_grp = lax.broadcasted_iota(jnp.int32, (_B_BLK, 128), 1) // 16
    h = dot(ux_, Wud[...]) + dot(sx_, Wsd[...]) + b0[...]
    S = dot(ux_, uw[...]) + dot(sx_, sw[...])
    ssq = (jnp.sum(ux_ * ux_ * uw2s[...], axis=1, keepdims=True)
           + jnp.sum(sx_ * sx_ * sw2s[...], axis=1, keepdims=True))
    for f in range(_F):
        for g_ref, rem_ref, W0x in ((gu, remu, W0a), (gs, rems, W0c)):
            gf = g_ref[pl.ds(f * _B_BLK, _B_BLK), :]
            rf = rem_ref[:, f][:, None]
            sel = dot(jnp.where(lane_grp == rf, gf, 0.0), R_)
            h = h + dot(sel, W0x[pl.ds(f * _D, _D), :])
            S = S + sel
            ssq = ssq + jnp.sum(sel * sel, axis=1, keepdims=True)
    h = _leaky(h)
    h = _leaky(dot(h, W1[...]) + b1[...])
    h = _leaky(dot(h, W2[...]) + b2[...])
    h = _leaky(dot(h, W3[...]) + b3[...])
    deep = dot(h, W4[...]) + b4p[...]
    sqsum = jnp.sum(S * S, axis=1, keepdims=True)
    v1 = (jnp.sum(ulg[...], axis=1, keepdims=True)
          + jnp.sum(slg[...], axis=1, keepdims=True)
          + jnp.sum(ux_ * udlw[...], axis=1, keepdims=True)
          + jnp.sum(sx_ * sdlw[...], axis=1, keepdims=True))
    out[...] = deep + v1 + 0.5 * (sqsum - ssq)


def kernel(user_sparse_x, user_dense_x, spu_sparse_x, spu_dense_x,
           user_table, spu_table, user_lin_table, spu_lin_table,
           user_dense_w, spu_dense_w, user_dense_lin_w, spu_dense_lin_w,
           fm_bias, W0, b0, W1, b1, W2, b2, W3, b3, W4, b4):
    f32 = jnp.float32
    off = (jnp.arange(_F, dtype=jnp.int32) * 100000)[None, :]
    idx_u = user_sparse_x.astype(jnp.int32) + off
    idx_s = spu_sparse_x.astype(jnp.int32) + off
    n_blk = _B // _B_BLK
    def pack_order(idx):
        q = (idx // 8).reshape(n_blk, _B_BLK, _F)
        return q.transpose(0, 2, 1).reshape(-1)
    pku = pack_order(idx_u)
    pks = pack_order(idx_s)
    rem_u = idx_u % 8
    rem_s = idx_s % 8
    ilu = idx_u.reshape(-1)
    ils = idx_s.reshape(-1)

    gu, gs, lu, ls = _sc_gather(
        pku, pks, ilu, ils,
        user_table.reshape(162500, 128),
        spu_table.reshape(162500, 128),
        user_lin_table.reshape(-1), spu_lin_table.reshape(-1))
    ulg = lu.reshape(_B, _F)
    slg = ls.reshape(_B, _F)

    uw = user_dense_w[0]
    sw = spu_dense_w[0]
    W0a = W0[:_H]
    Wud = jnp.einsum("fd,fdn->fn", uw, W0[_H:2 * _H].reshape(_F, _D, -1))
    W0c = W0[2 * _H:3 * _H]
    Wsd = jnp.einsum("fd,fdn->fn", sw, W0[3 * _H:4 * _H].reshape(_F, _D, -1))
    R = jnp.tile(jnp.eye(_D, dtype=f32), (8, 1))
    uw2s = jnp.sum(uw * uw, axis=1)[None, :]
    sw2s = jnp.sum(sw * sw, axis=1)[None, :]
    udlw = user_dense_lin_w[0, :, 0][None, :]
    sdlw = spu_dense_lin_w[0, :, 0][None, :]
    b4p = (b4 + fm_bias)[None, :]

    bspec_slab = pl.BlockSpec((_F * _B_BLK, 128), lambda i: (i, 0))
    bspec_batch = lambda n: pl.BlockSpec((_B_BLK, n), lambda i: (i, 0))
    bspec_w = lambda a: pl.BlockSpec(a.shape, lambda i: (0, 0))
    weights = [R, W0a, W0c, Wud, Wsd, uw, sw, uw2s, sw2s, udlw, sdlw,
               b0[None, :], W1, b1[None, :], W2, b2[None, :],
               W3, b3[None, :], W4, b4p]
    out = pl.pallas_call(
        _tc_body,
        grid=(n_blk,),
        in_specs=[bspec_slab, bspec_slab,
                  bspec_batch(_F), bspec_batch(_F), bspec_batch(_F),
                  bspec_batch(_F), bspec_batch(_F), bspec_batch(_F)]
                 + [bspec_w(a) for a in weights],
        out_specs=pl.BlockSpec((_B_BLK, 1), lambda i: (i, 0)),
        out_shape=jax.ShapeDtypeStruct((_B, 1), f32),
    )(gu, gs, rem_u, rem_s, ulg, slg, user_dense_x, spu_dense_x, *weights)
    return out[:, 0]


# speedup vs baseline: 1.0492x; 1.0492x over previous
import functools

import jax
import jax.numpy as jnp
from jax import lax
from jax.experimental import pallas as pl
from jax.experimental.pallas import tpu as pltpu
from jax.experimental.pallas import tpu_sc as plsc

_B = 16384
_F = 13
_D = 16
_NW = 32
_N = _B * _F
_RPW = _N // _NW
_ECH = 416
_NECH = _RPW // _ECH
_LCH = 832
_NLCH = _RPW // _LCH

_B_BLK = 512
_H = 208


def _sc_gather_body(pku, pks, ilu, ils, tabu, tabs, linu, lins,
                    gu, gs, lu, ls,
                    idx_v, rows_v, lidx_v, lval_v, sem):
    wid = lax.axis_index("s") * 2 + lax.axis_index("c")
    for c in range(_NECH):
        base = wid * _RPW + c * _ECH
        pltpu.sync_copy(pku.at[pl.ds(base, _ECH)], idx_v)
        pltpu.async_copy(tabu.at[idx_v], rows_v, sem).wait()
        pltpu.sync_copy(rows_v, gu.at[pl.ds(base, _ECH)])
        pltpu.sync_copy(pks.at[pl.ds(base, _ECH)], idx_v)
        pltpu.async_copy(tabs.at[idx_v], rows_v, sem).wait()
        pltpu.sync_copy(rows_v, gs.at[pl.ds(base, _ECH)])
    for c in range(_NLCH):
        base = wid * _RPW + c * _LCH
        pltpu.sync_copy(ilu.at[pl.ds(base, _LCH)], lidx_v)
        pltpu.async_copy(linu.at[lidx_v], lval_v, sem).wait()
        pltpu.sync_copy(lval_v, lu.at[pl.ds(base, _LCH)])
        pltpu.sync_copy(ils.at[pl.ds(base, _LCH)], lidx_v)
        pltpu.async_copy(lins.at[lidx_v], lval_v, sem).wait()
        pltpu.sync_copy(lval_v, ls.at[pl.ds(base, _LCH)])


def _sc_gather(pku, pks, ilu, ils, tabu, tabs, linu, lins):
    mesh = plsc.VectorSubcoreMesh(core_axis_name="c", subcore_axis_name="s")
    call = functools.partial(
        pl.kernel,
        mesh=mesh,
        out_type=[
            jax.ShapeDtypeStruct((_N, 128), jnp.float32),
            jax.ShapeDtypeStruct((_N, 128), jnp.float32),
            jax.ShapeDtypeStruct((_N,), jnp.float32),
            jax.ShapeDtypeStruct((_N,), jnp.float32),
        ],
        scratch_types=[
            pltpu.VMEM((_ECH,), jnp.int32),
            pltpu.VMEM((_ECH, 128), jnp.float32),
            pltpu.VMEM((_LCH,), jnp.int32),
            pltpu.VMEM((_LCH,), jnp.float32),
            pltpu.SemaphoreType.DMA,
        ],
    )(_sc_gather_body)
    return call(pku, pks, ilu, ils, tabu, tabs, linu, lins)


def _leaky(x):
    return jnp.where(x >= 0, x, 0.01 * x)


def _tc_body(gu, gs, remu, rems, ulg, slg, ux, sx,
             R, RWu, RWs, Wud, Wsd, uw, sw, uw2s, sw2s, udlw, sdlw,
             b0, W1, b1, W2, b2, W3, b3, W4, b4p, out):
    dot = lambda a, b: lax.dot_general(
        a, b, (((1,), (0,)), ((), ())), preferred_element_type=jnp.float32)
    ux_ = ux[...]
    sx_ = sx[...]
    lane_grp = lax.broadcasted_iota(jnp.int32, (_B_BLK, 128), 1) // 16
    h = dot(ux_, Wud[...]) + dot(sx_, Wsd[...]) + b0[...]
    mgsum = jnp.zeros((_B_BLK, 128), jnp.float32)
    ssqv = jnp.zeros((_B_BLK, 128), jnp.float32)
    for f in range(_F):
        for g_ref, rem_ref, RW in ((gu, remu, RWu), (gs, rems, RWs)):
            gf = g_ref[pl.ds(f * _B_BLK, _B_BLK), :]
            rf = rem_ref[:, f][:, None]
            mg = jnp.where(lane_grp == rf, gf, 0.0)
            h = h + dot(mg, RW[pl.ds(f * 128, 128), :])
            mgsum = mgsum + mg
            ssqv = ssqv + mg * gf
    S = dot(mgsum, R[...]) + dot(ux_, uw[...]) + dot(sx_, sw[...])
    ssq = (jnp.sum(ssqv, axis=1, keepdims=True)
           + jnp.sum(ux_ * ux_ * uw2s[...], axis=1, keepdims=True)
           + jnp.sum(sx_ * sx_ * sw2s[...], axis=1, keepdims=True))
    h = _leaky(h)
    h = _leaky(dot(h, W1[...]) + b1[...])
    h = _leaky(dot(h, W2[...]) + b2[...])
    h = _leaky(dot(h, W3[...]) + b3[...])
    deep = dot(h, W4[...]) + b4p[...]
    sqsum = jnp.sum(S * S, axis=1, keepdims=True)
    v1 = (jnp.sum(ulg[...], axis=1, keepdims=True)
          + jnp.sum(slg[...], axis=1, keepdims=True)
          + jnp.sum(ux_ * udlw[...], axis=1, keepdims=True)
          + jnp.sum(sx_ * sdlw[...], axis=1, keepdims=True))
    out[...] = deep + v1 + 0.5 * (sqsum - ssq)


def kernel(user_sparse_x, user_dense_x, spu_sparse_x, spu_dense_x,
           user_table, spu_table, user_lin_table, spu_lin_table,
           user_dense_w, spu_dense_w, user_dense_lin_w, spu_dense_lin_w,
           fm_bias, W0, b0, W1, b1, W2, b2, W3, b3, W4, b4):
    f32 = jnp.float32
    off = (jnp.arange(_F, dtype=jnp.int32) * 100000)[None, :]
    idx_u = user_sparse_x.astype(jnp.int32) + off
    idx_s = spu_sparse_x.astype(jnp.int32) + off
    n_blk = _B // _B_BLK
    def pack_order(idx):
        q = (idx // 8).reshape(n_blk, _B_BLK, _F)
        return q.transpose(0, 2, 1).reshape(-1)
    pku = pack_order(idx_u)
    pks = pack_order(idx_s)
    rem_u = idx_u % 8
    rem_s = idx_s % 8
    ilu = idx_u.reshape(-1)
    ils = idx_s.reshape(-1)

    gu, gs, lu, ls = _sc_gather(
        pku, pks, ilu, ils,
        user_table.reshape(162500, 128),
        spu_table.reshape(162500, 128),
        user_lin_table.reshape(-1), spu_lin_table.reshape(-1))
    ulg = lu.reshape(_B, _F)
    slg = ls.reshape(_B, _F)

    uw = user_dense_w[0]
    sw = spu_dense_w[0]
    Wud = jnp.einsum("fd,fdn->fn", uw, W0[_H:2 * _H].reshape(_F, _D, -1))
    Wsd = jnp.einsum("fd,fdn->fn", sw, W0[3 * _H:4 * _H].reshape(_F, _D, -1))
    R = jnp.tile(jnp.eye(_D, dtype=f32), (8, 1))
    RWu = jnp.tile(W0[:_H].reshape(_F, 1, _D, -1), (1, 8, 1, 1)).reshape(_F * 128, -1)
    RWs = jnp.tile(W0[2 * _H:3 * _H].reshape(_F, 1, _D, -1), (1, 8, 1, 1)).reshape(_F * 128, -1)
    uw2s = jnp.sum(uw * uw, axis=1)[None, :]
    sw2s = jnp.sum(sw * sw, axis=1)[None, :]
    udlw = user_dense_lin_w[0, :, 0][None, :]
    sdlw = spu_dense_lin_w[0, :, 0][None, :]
    b4p = (b4 + fm_bias)[None, :]

    bspec_slab = pl.BlockSpec((_F * _B_BLK, 128), lambda i: (i, 0))
    bspec_batch = lambda n: pl.BlockSpec((_B_BLK, n), lambda i: (i, 0))
    bspec_w = lambda a: pl.BlockSpec(a.shape, lambda i: (0, 0))
    weights = [R, RWu, RWs, Wud, Wsd, uw, sw, uw2s, sw2s, udlw, sdlw,
               b0[None, :], W1, b1[None, :], W2, b2[None, :],
               W3, b3[None, :], W4, b4p]
    out = pl.pallas_call(
        _tc_body,
        grid=(n_blk,),
        in_specs=[bspec_slab, bspec_slab,
                  bspec_batch(_F), bspec_batch(_F), bspec_batch(_F),
                  bspec_batch(_F), bspec_batch(_F), bspec_batch(_F)]
                 + [bspec_w(a) for a in weights],
        out_specs=pl.BlockSpec((_B_BLK, 1), lambda i: (i, 0)),
        out_shape=jax.ShapeDtypeStruct((_B, 1), f32),
    )(gu, gs, rem_u, rem_s, ulg, slg, user_dense_x, spu_dense_x, *weights)
    return out[:, 0]


# speedup vs baseline: 1.1885x; 1.1327x over previous
import functools

import jax
import jax.numpy as jnp
from jax import lax
from jax.experimental import pallas as pl
from jax.experimental.pallas import tpu as pltpu
from jax.experimental.pallas import tpu_sc as plsc

_B = 16384
_F = 13
_D = 16
_NW = 32
_N = _B * _F
_RPW = _N // _NW
_ECH = 416
_NECH = _RPW // _ECH
_LCH = 832
_NLCH = _RPW // _LCH

_B_BLK = 512
_H = 208


def _sc_gather_body(pku, pks, ilu, ils, tabu, tabs, linu, lins,
                    gu, gs, lu, ls,
                    idx_v, rows_v, lidx_v, lval_v, sem):
    wid = lax.axis_index("s") * 2 + lax.axis_index("c")
    for c in range(_NECH):
        base = wid * _RPW + c * _ECH
        pltpu.sync_copy(pku.at[pl.ds(base, _ECH)], idx_v)
        pltpu.async_copy(tabu.at[idx_v], rows_v, sem).wait()
        pltpu.sync_copy(rows_v, gu.at[pl.ds(base, _ECH)])
        pltpu.sync_copy(pks.at[pl.ds(base, _ECH)], idx_v)
        pltpu.async_copy(tabs.at[idx_v], rows_v, sem).wait()
        pltpu.sync_copy(rows_v, gs.at[pl.ds(base, _ECH)])
    for c in range(_NLCH):
        base = wid * _RPW + c * _LCH
        pltpu.sync_copy(ilu.at[pl.ds(base, _LCH)], lidx_v)
        pltpu.async_copy(linu.at[lidx_v], lval_v, sem).wait()
        pltpu.sync_copy(lval_v, lu.at[pl.ds(base, _LCH)])
        pltpu.sync_copy(ils.at[pl.ds(base, _LCH)], lidx_v)
        pltpu.async_copy(lins.at[lidx_v], lval_v, sem).wait()
        pltpu.sync_copy(lval_v, ls.at[pl.ds(base, _LCH)])


def _sc_gather(pku, pks, ilu, ils, tabu, tabs, linu, lins):
    mesh = plsc.VectorSubcoreMesh(core_axis_name="c", subcore_axis_name="s")
    call = functools.partial(
        pl.kernel,
        mesh=mesh,
        out_type=[
            jax.ShapeDtypeStruct((_N, 128), jnp.float32),
            jax.ShapeDtypeStruct((_N, 128), jnp.float32),
            jax.ShapeDtypeStruct((_N,), jnp.float32),
            jax.ShapeDtypeStruct((_N,), jnp.float32),
        ],
        scratch_types=[
            pltpu.VMEM((_ECH,), jnp.int32),
            pltpu.VMEM((_ECH, 128), jnp.float32),
            pltpu.VMEM((_LCH,), jnp.int32),
            pltpu.VMEM((_LCH,), jnp.float32),
            pltpu.SemaphoreType.DMA,
        ],
    )(_sc_gather_body)
    return call(pku, pks, ilu, ils, tabu, tabs, linu, lins)


def _leaky(x):
    return jnp.where(x >= 0, x, 0.01 * x)


def _tc_body(gu, gs, remu, rems, ulg, slg, ux, sx,
             R, RWu, RWs, Wud, Wsd, uw, sw, uw2s, sw2s, udlw, sdlw,
             b0, W1, b1, W2, b2, W3, b3, W4, b4p, out):
    dot = lambda a, b: lax.dot_general(
        a, b, (((1,), (0,)), ((), ())), preferred_element_type=jnp.float32)
    ux_ = ux[...]
    sx_ = sx[...]
    lane_grp = lax.broadcasted_iota(jnp.int32, (_B_BLK, 128), 1) // 16
    h = dot(ux_, Wud[...]) + dot(sx_, Wsd[...]) + b0[...]
    mgsum = jnp.zeros((_B_BLK, 128), jnp.float32)
    ssqv = jnp.zeros((_B_BLK, 128), jnp.float32)
    for f in range(_F):
        for g_ref, rem_ref, RW in ((gu, remu, RWu), (gs, rems, RWs)):
            gf = g_ref[pl.ds(f * _B_BLK, _B_BLK), :]
            rf = rem_ref[:, f][:, None]
            mg = jnp.where(lane_grp == rf, gf, 0.0)
            h = h + dot(mg, RW[pl.ds(f * 128, 128), :])
            mgsum = mgsum + mg
            ssqv = ssqv + mg * gf
    S = dot(mgsum, R[...]) + dot(ux_, uw[...]) + dot(sx_, sw[...])
    ssq = (jnp.sum(ssqv, axis=1, keepdims=True)
           + jnp.sum(ux_ * ux_ * uw2s[...], axis=1, keepdims=True)
           + jnp.sum(sx_ * sx_ * sw2s[...], axis=1, keepdims=True))
    h = _leaky(h)
    h = _leaky(dot(h, W1[...]) + b1[...])
    h = _leaky(dot(h, W2[...]) + b2[...])
    h = _leaky(dot(h, W3[...]) + b3[...])
    deep = dot(h, W4[...]) + b4p[...]
    sqsum = jnp.sum(S * S, axis=1, keepdims=True)
    v1 = (jnp.sum(ulg[...], axis=1, keepdims=True)
          + jnp.sum(slg[...], axis=1, keepdims=True)
          + jnp.sum(ux_ * udlw[...], axis=1, keepdims=True)
          + jnp.sum(sx_ * sdlw[...], axis=1, keepdims=True))
    out[...] = deep + v1 + 0.5 * (sqsum - ssq)


def kernel(user_sparse_x, user_dense_x, spu_sparse_x, spu_dense_x,
           user_table, spu_table, user_lin_table, spu_lin_table,
           user_dense_w, spu_dense_w, user_dense_lin_w, spu_dense_lin_w,
           fm_bias, W0, b0, W1, b1, W2, b2, W3, b3, W4, b4):
    f32 = jnp.float32
    off = (jnp.arange(_F, dtype=jnp.int32) * 100000)[None, :]
    idx_u = user_sparse_x.astype(jnp.int32) + off
    idx_s = spu_sparse_x.astype(jnp.int32) + off
    n_blk = _B // _B_BLK
    def pack_order(idx):
        q = (idx // 8).reshape(n_blk, _B_BLK, _F)
        return q.transpose(0, 2, 1).reshape(-1)
    pku = pack_order(idx_u)
    pks = pack_order(idx_s)
    rem_u = idx_u % 8
    rem_s = idx_s % 8
    ilu = idx_u.reshape(-1)
    ils = idx_s.reshape(-1)

    gu, gs, lu, ls = _sc_gather(
        pku, pks, ilu, ils,
        user_table.reshape(162500, 128),
        spu_table.reshape(162500, 128),
        user_lin_table.reshape(-1), spu_lin_table.reshape(-1))
    ulg = lu.reshape(_B, _F)
    slg = ls.reshape(_B, _F)

    s = (gu[1234, :16].sum() + gs[999, :16].sum()
         + ulg[:, 0].sum() + slg[:, 0].sum())
    return jnp.full((_B,), s, dtype=f32)


# speedup vs baseline: 1.4147x; 1.1904x over previous
import functools

import jax
import jax.numpy as jnp
from jax import lax
from jax.experimental import pallas as pl
from jax.experimental.pallas import tpu as pltpu
from jax.experimental.pallas import tpu_sc as plsc

_B = 16384
_F = 13
_D = 16
_NW = 32
_N = _B * _F
_RPW = _N // _NW
_ECH = 416
_NECH = _RPW // _ECH
_LCH = 832
_NLCH = _RPW // _LCH

_B_BLK = 512
_H = 208


def _sc_gather_body(pku, pks, ilu, ils, tabu, tabs, linu, lins,
                    gu, gs, lu, ls,
                    idx_v, rows_v, lidx_v, lval_v, sem):
    wid = lax.axis_index("s") * 2 + lax.axis_index("c")
    for c in range(_NECH):
        base = wid * _RPW + c * _ECH
        pltpu.sync_copy(pku.at[pl.ds(base, _ECH)], idx_v)
        pltpu.async_copy(tabu.at[idx_v], rows_v, sem).wait()
        pltpu.sync_copy(rows_v, gu.at[pl.ds(base, _ECH)])
        pltpu.sync_copy(pks.at[pl.ds(base, _ECH)], idx_v)
        pltpu.async_copy(tabs.at[idx_v], rows_v, sem).wait()
        pltpu.sync_copy(rows_v, gs.at[pl.ds(base, _ECH)])
    for c in range(_NLCH):
        base = wid * _RPW + c * _LCH
        pltpu.sync_copy(ilu.at[pl.ds(base, _LCH)], lidx_v)
        pltpu.async_copy(linu.at[lidx_v], lval_v, sem).wait()
        pltpu.sync_copy(lval_v, lu.at[pl.ds(base, _LCH)])
        pltpu.sync_copy(ils.at[pl.ds(base, _LCH)], lidx_v)
        pltpu.async_copy(lins.at[lidx_v], lval_v, sem).wait()
        pltpu.sync_copy(lval_v, ls.at[pl.ds(base, _LCH)])


def _sc_gather(pku, pks, ilu, ils, tabu, tabs, linu, lins):
    mesh = plsc.VectorSubcoreMesh(core_axis_name="c", subcore_axis_name="s")
    call = functools.partial(
        pl.kernel,
        mesh=mesh,
        out_type=[
            jax.ShapeDtypeStruct((_N, 128), jnp.float32),
            jax.ShapeDtypeStruct((_N, 128), jnp.float32),
            jax.ShapeDtypeStruct((_N,), jnp.float32),
            jax.ShapeDtypeStruct((_N,), jnp.float32),
        ],
        scratch_types=[
            pltpu.VMEM((_ECH,), jnp.int32),
            pltpu.VMEM((_ECH, 128), jnp.float32),
            pltpu.VMEM((_LCH,), jnp.int32),
            pltpu.VMEM((_LCH,), jnp.float32),
            pltpu.SemaphoreType.DMA,
        ],
    )(_sc_gather_body)
    return call(pku, pks, ilu, ils, tabu, tabs, linu, lins)


def _pack128(tab):
    return tab.T.reshape(_D, 162500, 8).transpose(1, 2, 0).reshape(162500, 128)


def _leaky(x):
    return jnp.where(x >= 0, x, 0.01 * x)


def _tc_body(gu, gs, remu, rems, ulg, slg, ux, sx,
             R, RWu, RWs, Wud, Wsd, uw, sw, uw2s, sw2s, udlw, sdlw,
             b0, W1, b1, W2, b2, W3, b3, W4, b4p, out):
    dot = lambda a, b: lax.dot_general(
        a, b, (((1,), (0,)), ((), ())), preferred_element_type=jnp.float32)
    ux_ = ux[...]
    sx_ = sx[...]
    lane_grp = lax.broadcasted_iota(jnp.int32, (_B_BLK, 128), 1) // 16
    h = dot(ux_, Wud[...]) + dot(sx_, Wsd[...]) + b0[...]
    mgsum = jnp.zeros((_B_BLK, 128), jnp.float32)
    ssqv = jnp.zeros((_B_BLK, 128), jnp.float32)
    for f in range(_F):
        for g_ref, rem_ref, RW in ((gu, remu, RWu), (gs, rems, RWs)):
            gf = g_ref[pl.ds(f * _B_BLK, _B_BLK), :]
            rf = rem_ref[:, f][:, None]
            mg = jnp.where(lane_grp == rf, gf, 0.0)
            h = h + dot(mg, RW[pl.ds(f * 128, 128), :])
            mgsum = mgsum + mg
            ssqv = ssqv + mg * gf
    S = dot(mgsum, R[...]) + dot(ux_, uw[...]) + dot(sx_, sw[...])
    ssq = (jnp.sum(ssqv, axis=1, keepdims=True)
           + jnp.sum(ux_ * ux_ * uw2s[...], axis=1, keepdims=True)
           + jnp.sum(sx_ * sx_ * sw2s[...], axis=1, keepdims=True))
    h = _leaky(h)
    h = _leaky(dot(h, W1[...]) + b1[...])
    h = _leaky(dot(h, W2[...]) + b2[...])
    h = _leaky(dot(h, W3[...]) + b3[...])
    deep = dot(h, W4[...]) + b4p[...]
    sqsum = jnp.sum(S * S, axis=1, keepdims=True)
    v1 = (jnp.sum(ulg[...], axis=1, keepdims=True)
          + jnp.sum(slg[...], axis=1, keepdims=True)
          + jnp.sum(ux_ * udlw[...], axis=1, keepdims=True)
          + jnp.sum(sx_ * sdlw[...], axis=1, keepdims=True))
    out[...] = deep + v1 + 0.5 * (sqsum - ssq)


def kernel(user_sparse_x, user_dense_x, spu_sparse_x, spu_dense_x,
           user_table, spu_table, user_lin_table, spu_lin_table,
           user_dense_w, spu_dense_w, user_dense_lin_w, spu_dense_lin_w,
           fm_bias, W0, b0, W1, b1, W2, b2, W3, b3, W4, b4):
    f32 = jnp.float32
    off = (jnp.arange(_F, dtype=jnp.int32) * 100000)[None, :]
    idx_u = user_sparse_x.astype(jnp.int32) + off
    idx_s = spu_sparse_x.astype(jnp.int32) + off
    n_blk = _B // _B_BLK
    def pack_order(idx):
        q = (idx // 8).reshape(n_blk, _B_BLK, _F)
        return q.transpose(0, 2, 1).reshape(-1)
    pku = pack_order(idx_u)
    pks = pack_order(idx_s)
    rem_u = idx_u % 8
    rem_s = idx_s % 8
    ilu = idx_u.reshape(-1)
    ils = idx_s.reshape(-1)

    gu, gs, lu, ls = _sc_gather(
        pku, pks, ilu, ils,
        _pack128(user_table),
        _pack128(spu_table),
        user_lin_table.reshape(-1), spu_lin_table.reshape(-1))
    ulg = lu.reshape(_B, _F)
    slg = ls.reshape(_B, _F)

    uw = user_dense_w[0]
    sw = spu_dense_w[0]
    Wud = jnp.einsum("fd,fdn->fn", uw, W0[_H:2 * _H].reshape(_F, _D, -1))
    Wsd = jnp.einsum("fd,fdn->fn", sw, W0[3 * _H:4 * _H].reshape(_F, _D, -1))
    R = jnp.tile(jnp.eye(_D, dtype=f32), (8, 1))
    RWu = jnp.tile(W0[:_H].reshape(_F, 1, _D, -1), (1, 8, 1, 1)).reshape(_F * 128, -1)
    RWs = jnp.tile(W0[2 * _H:3 * _H].reshape(_F, 1, _D, -1), (1, 8, 1, 1)).reshape(_F * 128, -1)
    uw2s = jnp.sum(uw * uw, axis=1)[None, :]
    sw2s = jnp.sum(sw * sw, axis=1)[None, :]
    udlw = user_dense_lin_w[0, :, 0][None, :]
    sdlw = spu_dense_lin_w[0, :, 0][None, :]
    b4p = (b4 + fm_bias)[None, :]

    bspec_slab = pl.BlockSpec((_F * _B_BLK, 128), lambda i: (i, 0))
    bspec_batch = lambda n: pl.BlockSpec((_B_BLK, n), lambda i: (i, 0))
    bspec_w = lambda a: pl.BlockSpec(a.shape, lambda i: (0, 0))
    weights = [R, RWu, RWs, Wud, Wsd, uw, sw, uw2s, sw2s, udlw, sdlw,
               b0[None, :], W1, b1[None, :], W2, b2[None, :],
               W3, b3[None, :], W4, b4p]
    out = pl.pallas_call(
        _tc_body,
        grid=(n_blk,),
        in_specs=[bspec_slab, bspec_slab,
                  bspec_batch(_F), bspec_batch(_F), bspec_batch(_F),
                  bspec_batch(_F), bspec_batch(_F), bspec_batch(_F)]
                 + [bspec_w(a) for a in weights],
        out_specs=pl.BlockSpec((_B_BLK, 1), lambda i: (i, 0)),
        out_shape=jax.ShapeDtypeStruct((_B, 1), f32),
    )(gu, gs, rem_u, rem_s, ulg, slg, user_dense_x, spu_dense_x, *weights)
    return out[:, 0]


# speedup vs baseline: 1.4209x; 1.0044x over previous
import functools

import jax
import jax.numpy as jnp
from jax import lax
from jax.experimental import pallas as pl
from jax.experimental.pallas import tpu as pltpu
from jax.experimental.pallas import tpu_sc as plsc

_B = 16384
_F = 13
_D = 16
_NW = 32
_N = _B * _F
_RPW = _N // _NW
_ECH = 416
_NECH = _RPW // _ECH
_LCH = 832
_NLCH = _RPW // _LCH

_B_BLK = 512
_H = 208


def _sc_gather_body(pku, pks, ilu, ils, tabu, tabs, linu, lins,
                    gu, gs, lu, ls,
                    idx_v, rows_v, lidx_v, lval_v, sem):
    wid = lax.axis_index("s") * 2 + lax.axis_index("c")
    for c in range(_NECH):
        base = wid * _RPW + c * _ECH
        pltpu.sync_copy(pku.at[pl.ds(base, _ECH)], idx_v)
        pltpu.async_copy(tabu.at[idx_v], rows_v, sem).wait()
        pltpu.sync_copy(rows_v, gu.at[pl.ds(base, _ECH)])
        pltpu.sync_copy(pks.at[pl.ds(base, _ECH)], idx_v)
        pltpu.async_copy(tabs.at[idx_v], rows_v, sem).wait()
        pltpu.sync_copy(rows_v, gs.at[pl.ds(base, _ECH)])
    for c in range(_NLCH):
        base = wid * _RPW + c * _LCH
        pltpu.sync_copy(ilu.at[pl.ds(base, _LCH)], lidx_v)
        pltpu.async_copy(linu.at[lidx_v], lval_v, sem).wait()
        pltpu.sync_copy(lval_v, lu.at[pl.ds(base, _LCH)])
        pltpu.sync_copy(ils.at[pl.ds(base, _LCH)], lidx_v)
        pltpu.async_copy(lins.at[lidx_v], lval_v, sem).wait()
        pltpu.sync_copy(lval_v, ls.at[pl.ds(base, _LCH)])


def _sc_gather(pku, pks, ilu, ils, tabu, tabs, linu, lins):
    mesh = plsc.VectorSubcoreMesh(core_axis_name="c", subcore_axis_name="s")
    call = functools.partial(
        pl.kernel,
        mesh=mesh,
        out_type=[
            jax.ShapeDtypeStruct((_N, 128), jnp.float32),
            jax.ShapeDtypeStruct((_N, 128), jnp.float32),
            jax.ShapeDtypeStruct((_N,), jnp.float32),
            jax.ShapeDtypeStruct((_N,), jnp.float32),
        ],
        scratch_types=[
            pltpu.VMEM((_ECH,), jnp.int32),
            pltpu.VMEM((_ECH, 128), jnp.float32),
            pltpu.VMEM((_LCH,), jnp.int32),
            pltpu.VMEM((_LCH,), jnp.float32),
            pltpu.SemaphoreType.DMA,
        ],
    )(_sc_gather_body)
    return call(pku, pks, ilu, ils, tabu, tabs, linu, lins)


def _pack128(tab):
    return tab.T.reshape(_D, 162500, 8).transpose(1, 2, 0).reshape(162500, 128)


def _leaky(x):
    return jnp.where(x >= 0, x, 0.01 * x)


def _tc_body(gu, gs, remu, rems, ulg, slg, ux, sx,
             R, RWu, RWs, Wud, Wsd, uw, sw, uw2s, sw2s, udlw, sdlw,
             b0, W1, b1, W2, b2, W3, b3, W4, b4p, out):
    dot = lambda a, b: lax.dot_general(
        a, b, (((1,), (0,)), ((), ())), preferred_element_type=jnp.float32)
    ux_ = ux[...]
    sx_ = sx[...]
    lane_grp = lax.broadcasted_iota(jnp.int32, (_B_BLK, 128), 1) // 16
    h = dot(ux_, Wud[...]) + dot(sx_, Wsd[...]) + b0[...]
    mgsum = jnp.zeros((_B_BLK, 128), jnp.float32)
    ssqv = jnp.zeros((_B_BLK, 128), jnp.float32)
    for f in range(_F):
        for g_ref, rem_ref, RW in ((gu, remu, RWu), (gs, rems, RWs)):
            gf = g_ref[pl.ds(f * _B_BLK, _B_BLK), :]
            rf = rem_ref[:, f][:, None]
            mg = jnp.where(lane_grp == rf, gf, 0.0)
            h = h + dot(mg, RW[pl.ds(f * 128, 128), :])
            mgsum = mgsum + mg
            ssqv = ssqv + mg * gf
    S = dot(mgsum, R[...]) + dot(ux_, uw[...]) + dot(sx_, sw[...])
    ssq = (jnp.sum(ssqv, axis=1, keepdims=True)
           + jnp.sum(ux_ * ux_ * uw2s[...], axis=1, keepdims=True)
           + jnp.sum(sx_ * sx_ * sw2s[...], axis=1, keepdims=True))
    h = _leaky(h)
    h = _leaky(dot(h, W1[...]) + b1[...])
    h = _leaky(dot(h, W2[...]) + b2[...])
    h = _leaky(dot(h, W3[...]) + b3[...])
    deep = dot(h, W4[...]) + b4p[...]
    sqsum = jnp.sum(S * S, axis=1, keepdims=True)
    v1 = (jnp.sum(ulg[...], axis=1, keepdims=True)
          + jnp.sum(slg[...], axis=1, keepdims=True)
          + jnp.sum(ux_ * udlw[...], axis=1, keepdims=True)
          + jnp.sum(sx_ * sdlw[...], axis=1, keepdims=True))
    out[...] = deep + v1 + 0.5 * (sqsum - ssq)


def kernel(user_sparse_x, user_dense_x, spu_sparse_x, spu_dense_x,
           user_table, spu_table, user_lin_table, spu_lin_table,
           user_dense_w, spu_dense_w, user_dense_lin_w, spu_dense_lin_w,
           fm_bias, W0, b0, W1, b1, W2, b2, W3, b3, W4, b4):
    f32 = jnp.float32
    off = (jnp.arange(_F, dtype=jnp.int32) * 100000)[None, :]
    idx_u = user_sparse_x.astype(jnp.int32) + off
    idx_s = spu_sparse_x.astype(jnp.int32) + off
    n_blk = _B // _B_BLK
    def pack_order(idx):
        q = (idx // 8).reshape(n_blk, _B_BLK, _F)
        return q.transpose(0, 2, 1).reshape(-1)
    pku = pack_order(idx_u)
    pks = pack_order(idx_s)
    rem_u = idx_u % 8
    rem_s = idx_s % 8
    ilu = idx_u.reshape(-1)
    ils = idx_s.reshape(-1)

    gu, gs, lu, ls = _sc_gather(
        pku, pks, ilu, ils,
        _pack128(user_table),
        _pack128(spu_table),
        user_lin_table[:, 0], spu_lin_table[:, 0])
    ulg = lu.reshape(_B, _F)
    slg = ls.reshape(_B, _F)

    uw = user_dense_w[0]
    sw = spu_dense_w[0]
    Wud = jnp.einsum("fd,fdn->fn", uw, W0[_H:2 * _H].reshape(_F, _D, -1))
    Wsd = jnp.einsum("fd,fdn->fn", sw, W0[3 * _H:4 * _H].reshape(_F, _D, -1))
    R = jnp.tile(jnp.eye(_D, dtype=f32), (8, 1))
    RWu = jnp.tile(W0[:_H].reshape(_F, 1, _D, -1), (1, 8, 1, 1)).reshape(_F * 128, -1)
    RWs = jnp.tile(W0[2 * _H:3 * _H].reshape(_F, 1, _D, -1), (1, 8, 1, 1)).reshape(_F * 128, -1)
    uw2s = jnp.sum(uw * uw, axis=1)[None, :]
    sw2s = jnp.sum(sw * sw, axis=1)[None, :]
    udlw = user_dense_lin_w[0, :, 0][None, :]
    sdlw = spu_dense_lin_w[0, :, 0][None, :]
    b4p = (b4 + fm_bias)[None, :]

    bspec_slab = pl.BlockSpec((_F * _B_BLK, 128), lambda i: (i, 0))
    bspec_batch = lambda n: pl.BlockSpec((_B_BLK, n), lambda i: (i, 0))
    bspec_w = lambda a: pl.BlockSpec(a.shape, lambda i: (0, 0))
    weights = [R, RWu, RWs, Wud, Wsd, uw, sw, uw2s, sw2s, udlw, sdlw,
               b0[None, :], W1, b1[None, :], W2, b2[None, :],
               W3, b3[None, :], W4, b4p]
    out = pl.pallas_call(
        _tc_body,
        grid=(n_blk,),
        in_specs=[bspec_slab, bspec_slab,
                  bspec_batch(_F), bspec_batch(_F), bspec_batch(_F),
                  bspec_batch(_F), bspec_batch(_F), bspec_batch(_F)]
                 + [bspec_w(a) for a in weights],
        out_specs=pl.BlockSpec((_B_BLK, 1), lambda i: (i, 0)),
        out_shape=jax.ShapeDtypeStruct((_B, 1), f32),
    )(gu, gs, rem_u, rem_s, ulg, slg, user_dense_x, spu_dense_x, *weights)
    return out[:, 0]


# speedup vs baseline: 1.4648x; 1.0309x over previous
import functools

import jax
import jax.numpy as jnp
from jax import lax
from jax.experimental import pallas as pl
from jax.experimental.pallas import tpu as pltpu
from jax.experimental.pallas import tpu_sc as plsc

_B = 16384
_F = 13
_D = 16
_NW = 32
_N = _B * _F
_RPW = _N // _NW
_ECH = 416
_NECH = _RPW // _ECH
_LCH = 832
_NLCH = _RPW // _LCH

_B_BLK = 512
_H = 208


def _sc_gather_body(pku, pks, ilu, ils, tabu, tabs, linu, lins,
                    gu, gs, lu, ls,
                    idx_v0, rows_v0, idx_v1, rows_v1, lidx_v, lval_v,
                    sem0, sem1):
    wid = lax.axis_index("s") * 2 + lax.axis_index("c")
    idx_b = (idx_v0, idx_v1)
    rows_b = (rows_v0, rows_v1)
    sem_b = (sem0, sem1)
    nslots = 2 * _NECH

    def slot_args(t):
        tab = tabu if t % 2 == 0 else tabs
        src = pku if t % 2 == 0 else pks
        dst = gu if t % 2 == 0 else gs
        base = wid * _RPW + (t // 2) * _ECH
        return tab, src, dst, base

    tab, src, dst, base = slot_args(0)
    pltpu.sync_copy(src.at[pl.ds(base, _ECH)], idx_b[0])
    cp_prev = pltpu.async_copy(tab.at[idx_b[0]], rows_b[0], sem_b[0])
    for t in range(nslots):
        k = t % 2
        nk = (t + 1) % 2
        if t + 1 < nslots:
            tab, src, dst_n, base_n = slot_args(t + 1)
            pltpu.sync_copy(src.at[pl.ds(base_n, _ECH)], idx_b[nk])
            cp_next = pltpu.async_copy(tab.at[idx_b[nk]], rows_b[nk], sem_b[nk])
        cp_prev.wait()
        _, _, dst, base = slot_args(t)
        pltpu.sync_copy(rows_b[k], dst.at[pl.ds(base, _ECH)])
        if t + 1 < nslots:
            cp_prev = cp_next
    for c in range(_NLCH):
        base = wid * _RPW + c * _LCH
        pltpu.sync_copy(ilu.at[pl.ds(base, _LCH)], lidx_v)
        pltpu.async_copy(linu.at[lidx_v], lval_v, sem0).wait()
        pltpu.sync_copy(lval_v, lu.at[pl.ds(base, _LCH)])
        pltpu.sync_copy(ils.at[pl.ds(base, _LCH)], lidx_v)
        pltpu.async_copy(lins.at[lidx_v], lval_v, sem0).wait()
        pltpu.sync_copy(lval_v, ls.at[pl.ds(base, _LCH)])


def _sc_gather(pku, pks, ilu, ils, tabu, tabs, linu, lins):
    mesh = plsc.VectorSubcoreMesh(core_axis_name="c", subcore_axis_name="s")
    call = functools.partial(
        pl.kernel,
        mesh=mesh,
        out_type=[
            jax.ShapeDtypeStruct((_N, 128), jnp.float32),
            jax.ShapeDtypeStruct((_N, 128), jnp.float32),
            jax.ShapeDtypeStruct((_N,), jnp.float32),
            jax.ShapeDtypeStruct((_N,), jnp.float32),
        ],
        scratch_types=[
            pltpu.VMEM((_ECH,), jnp.int32),
            pltpu.VMEM((_ECH, 128), jnp.float32),
            pltpu.VMEM((_ECH,), jnp.int32),
            pltpu.VMEM((_ECH, 128), jnp.float32),
            pltpu.VMEM((_LCH,), jnp.int32),
            pltpu.VMEM((_LCH,), jnp.float32),
            pltpu.SemaphoreType.DMA,
            pltpu.SemaphoreType.DMA,
        ],
    )(_sc_gather_body)
    return call(pku, pks, ilu, ils, tabu, tabs, linu, lins)


def _pack128(tab):
    return tab.T.reshape(_D, 162500, 8).transpose(1, 2, 0).reshape(162500, 128)


def _leaky(x):
    return jnp.where(x >= 0, x, 0.01 * x)


def _tc_body(gu, gs, remu, rems, ulg, slg, ux, sx,
             R, RWu, RWs, Wud, Wsd, uw, sw, uw2s, sw2s, udlw, sdlw,
             b0, W1, b1, W2, b2, W3, b3, W4, b4p, out):
    dot = lambda a, b: lax.dot_general(
        a, b, (((1,), (0,)), ((), ())), preferred_element_type=jnp.float32)
    ux_ = ux[...]
    sx_ = sx[...]
    lane_grp = lax.broadcasted_iota(jnp.int32, (_B_BLK, 128), 1) // 16
    h = dot(ux_, Wud[...]) + dot(sx_, Wsd[...]) + b0[...]
    mgsum = jnp.zeros((_B_BLK, 128), jnp.float32)
    ssqv = jnp.zeros((_B_BLK, 128), jnp.float32)
    for f in range(_F):
        for g_ref, rem_ref, RW in ((gu, remu, RWu), (gs, rems, RWs)):
            gf = g_ref[pl.ds(f * _B_BLK, _B_BLK), :]
            rf = rem_ref[:, f][:, None]
            mg = jnp.where(lane_grp == rf, gf, 0.0)
            h = h + dot(mg, RW[pl.ds(f * 128, 128), :])
            mgsum = mgsum + mg
            ssqv = ssqv + mg * gf
    S = dot(mgsum, R[...]) + dot(ux_, uw[...]) + dot(sx_, sw[...])
    ssq = (jnp.sum(ssqv, axis=1, keepdims=True)
           + jnp.sum(ux_ * ux_ * uw2s[...], axis=1, keepdims=True)
           + jnp.sum(sx_ * sx_ * sw2s[...], axis=1, keepdims=True))
    h = _leaky(h)
    h = _leaky(dot(h, W1[...]) + b1[...])
    h = _leaky(dot(h, W2[...]) + b2[...])
    h = _leaky(dot(h, W3[...]) + b3[...])
    deep = dot(h, W4[...]) + b4p[...]
    sqsum = jnp.sum(S * S, axis=1, keepdims=True)
    v1 = (jnp.sum(ulg[...], axis=1, keepdims=True)
          + jnp.sum(slg[...], axis=1, keepdims=True)
          + jnp.sum(ux_ * udlw[...], axis=1, keepdims=True)
          + jnp.sum(sx_ * sdlw[...], axis=1, keepdims=True))
    out[...] = deep + v1 + 0.5 * (sqsum - ssq)


def kernel(user_sparse_x, user_dense_x, spu_sparse_x, spu_dense_x,
           user_table, spu_table, user_lin_table, spu_lin_table,
           user_dense_w, spu_dense_w, user_dense_lin_w, spu_dense_lin_w,
           fm_bias, W0, b0, W1, b1, W2, b2, W3, b3, W4, b4):
    f32 = jnp.float32
    off = (jnp.arange(_F, dtype=jnp.int32) * 100000)[None, :]
    idx_u = user_sparse_x.astype(jnp.int32) + off
    idx_s = spu_sparse_x.astype(jnp.int32) + off
    n_blk = _B // _B_BLK
    def pack_order(idx):
        q = (idx // 8).reshape(n_blk, _B_BLK, _F)
        return q.transpose(0, 2, 1).reshape(-1)
    pku = pack_order(idx_u)
    pks = pack_order(idx_s)
    rem_u = idx_u % 8
    rem_s = idx_s % 8
    ilu = idx_u.reshape(-1)
    ils = idx_s.reshape(-1)

    gu, gs, lu, ls = _sc_gather(
        pku, pks, ilu, ils,
        _pack128(user_table),
        _pack128(spu_table),
        user_lin_table[:, 0], spu_lin_table[:, 0])
    ulg = lu.reshape(_B, _F)
    slg = ls.reshape(_B, _F)

    uw = user_dense_w[0]
    sw = spu_dense_w[0]
    Wud = jnp.einsum("fd,fdn->fn", uw, W0[_H:2 * _H].reshape(_F, _D, -1))
    Wsd = jnp.einsum("fd,fdn->fn", sw, W0[3 * _H:4 * _H].reshape(_F, _D, -1))
    R = jnp.tile(jnp.eye(_D, dtype=f32), (8, 1))
    RWu = jnp.tile(W0[:_H].reshape(_F, 1, _D, -1), (1, 8, 1, 1)).reshape(_F * 128, -1)
    RWs = jnp.tile(W0[2 * _H:3 * _H].reshape(_F, 1, _D, -1), (1, 8, 1, 1)).reshape(_F * 128, -1)
    uw2s = jnp.sum(uw * uw, axis=1)[None, :]
    sw2s = jnp.sum(sw * sw, axis=1)[None, :]
    udlw = user_dense_lin_w[0, :, 0][None, :]
    sdlw = spu_dense_lin_w[0, :, 0][None, :]
    b4p = (b4 + fm_bias)[None, :]

    bspec_slab = pl.BlockSpec((_F * _B_BLK, 128), lambda i: (i, 0))
    bspec_batch = lambda n: pl.BlockSpec((_B_BLK, n), lambda i: (i, 0))
    bspec_w = lambda a: pl.BlockSpec(a.shape, lambda i: (0, 0))
    weights = [R, RWu, RWs, Wud, Wsd, uw, sw, uw2s, sw2s, udlw, sdlw,
               b0[None, :], W1, b1[None, :], W2, b2[None, :],
               W3, b3[None, :], W4, b4p]
    out = pl.pallas_call(
        _tc_body,
        grid=(n_blk,),
        in_specs=[bspec_slab, bspec_slab,
                  bspec_batch(_F), bspec_batch(_F), bspec_batch(_F),
                  bspec_batch(_F), bspec_batch(_F), bspec_batch(_F)]
                 + [bspec_w(a) for a in weights],
        out_specs=pl.BlockSpec((_B_BLK, 1), lambda i: (i, 0)),
        out_shape=jax.ShapeDtypeStruct((_B, 1), f32),
    )(gu, gs, rem_u, rem_s, ulg, slg, user_dense_x, spu_dense_x, *weights)
    return out[:, 0]
